# baseline (speedup 1.0000x reference)
import numpy as np
import jax
import jax.numpy as jnp
from jax import lax
from jax.experimental import pallas as pl

_V = 100000
_R = 8
_NROWS = 512
_CH = 256
_NMAIN = 390
_REM_OFF = 99840
_TAIL_OFF = 99968

_ROT = ((13, 15, 26, 6), (17, 29, 16, 24),
        (13, 15, 26, 6), (17, 29, 16, 24),
        (13, 15, 26, 6))

_KS = (np.uint32(0), np.uint32(42),
       np.uint32(np.uint32(42) ^ np.uint32(0x1BD11BDA)))

_TINY = np.float32(np.finfo(np.float32).tiny)
_NEGINF = np.float32(-np.inf)


def _rotl(x, r):
    return (x << np.uint32(r)) | (x >> np.uint32(32 - r))


def _gumbel(flat_u32, lg_c):
    t0 = flat_u32 + _KS[1]
    x0 = t0
    x1 = _rotl(t0, 13) ^ t0
    first = True
    for g in range(5):
        for r in _ROT[g][1 if first else 0:]:
            x0 = x0 + x1
            x1 = _rotl(x1, r) ^ x0
        first = False
        x0 = x0 + _KS[(g + 1) % 3]
        x1 = x1 + _KS[(g + 2) % 3] + np.uint32(g + 1)
    bits = x0 ^ x1
    fl = lax.bitcast_convert_type(
        (bits >> np.uint32(9)) | np.uint32(0x3F800000), jnp.float32) - 1.0
    u = jnp.maximum(fl + _TINY, _TINY)
    return -jnp.log(-jnp.log(u)) + lg_c


def _body(lg_ref, samp_ref, lp_ref):
    i = pl.program_id(0)
    rowbase = i * _R

    def iotas(w):
        lane = lax.broadcasted_iota(jnp.int32, (_R, w), 1)
        row = lax.broadcasted_iota(jnp.int32, (_R, w), 0) + rowbase
        return lane, (row * _V + lane).astype(jnp.uint32)

    lane_m, flat0_m = iotas(_CH)

    def loop1(c, carry):
        acc_tv, acc_col, acc_lm = carry
        off = c * _CH
        lg_c = lg_ref[:, pl.ds(off, _CH)]
        t = _gumbel(flat0_m + jnp.uint32(off), lg_c)
        upd = t > acc_tv
        acc_col = jnp.where(upd, lane_m + off, acc_col)
        acc_tv = jnp.maximum(acc_tv, t)
        acc_lm = jnp.maximum(acc_lm, lg_c)
        return acc_tv, acc_col, acc_lm

    acc_tv, acc_col, acc_lm = lax.fori_loop(
        0, _NMAIN, loop1,
        (jnp.full((_R, _CH), _NEGINF),
         jnp.zeros((_R, _CH), jnp.int32),
         jnp.full((_R, _CH), _NEGINF)))

    def piece(off, w):
        lane, flat0 = iotas(w)
        lg_c = lg_ref[:, pl.ds(off, w)]
        t = _gumbel(flat0 + jnp.uint32(off), lg_c)
        return t, lg_c, lane + off

    t_rem, lg_rem, col_rem = piece(_REM_OFF, 128)
    t_tail, lg_tail, col_tail = piece(_TAIL_OFF, 32)

    red_max = lambda x: jnp.max(x, axis=-1, keepdims=True)
    red_sum = lambda x: jnp.sum(x, axis=-1, keepdims=True)

    m_t = jnp.maximum(jnp.maximum(red_max(acc_tv), red_max(t_rem)),
                      red_max(t_tail))
    cand = lambda t, c: jnp.min(jnp.where(t == m_t, c, _V), axis=-1,
                                keepdims=True)
    samp = jnp.minimum(jnp.minimum(cand(acc_tv, acc_col),
                                   cand(t_rem, col_rem)),
                       cand(t_tail, col_tail))

    m_l = jnp.maximum(jnp.maximum(red_max(acc_lm), red_max(lg_rem)),
                      red_max(lg_tail))

    def loop2(c, carry):
        acc_s, acc_ch = carry
        off = c * _CH
        lg_c = lg_ref[:, pl.ds(off, _CH)]
        acc_s = acc_s + jnp.exp(lg_c - m_l)
        acc_ch = acc_ch + jnp.where(lane_m + off == samp, lg_c, 0.0)
        return acc_s, acc_ch

    acc_s, acc_ch = lax.fori_loop(
        0, _NMAIN, loop2,
        (jnp.zeros((_R, _CH), jnp.float32),
         jnp.zeros((_R, _CH), jnp.float32)))

    s = (red_sum(acc_s) + red_sum(jnp.exp(lg_rem - m_l))
         + red_sum(jnp.exp(lg_tail - m_l)))
    chosen = (red_sum(acc_ch)
              + red_sum(jnp.where(col_rem == samp, lg_rem, 0.0))
              + red_sum(jnp.where(col_tail == samp, lg_tail, 0.0)))
    lp_row = (chosen - m_l) - jnp.log(s)

    samp_ref[0] = samp
    lp_ref[0] = jnp.full((_R, 1), jnp.sum(lp_row), jnp.float32)


def kernel(logits):
    lg = logits.reshape(_NROWS, _V)
    nblk = _NROWS // _R
    samp, lp = pl.pallas_call(
        _body,
        grid=(nblk,),
        in_specs=[pl.BlockSpec((_R, _V), lambda i: (i, 0))],
        out_specs=[
            pl.BlockSpec((1, _R, 1), lambda i: (i, 0, 0)),
            pl.BlockSpec((1, _R, 1), lambda i: (i, 0, 0)),
        ],
        out_shape=[
            jax.ShapeDtypeStruct((nblk, _R, 1), jnp.int32),
            jax.ShapeDtypeStruct((nblk, _R, 1), jnp.float32),
        ],
    )(lg)
    sample = samp[..., 0]
    log_prob = lp[:, 0, 0]
    return sample, log_prob


# speedup vs baseline: 1.7748x; 1.7748x over previous
import numpy as np
import jax
import jax.numpy as jnp
from jax import lax
from jax.experimental import pallas as pl

_V = 100000
_R = 8
_NROWS = 512

_ROT = ((13, 15, 26, 6), (17, 29, 16, 24),
        (13, 15, 26, 6), (17, 29, 16, 24),
        (13, 15, 26, 6))

_KS = (np.uint32(0), np.uint32(42),
       np.uint32(np.uint32(42) ^ np.uint32(0x1BD11BDA)))

_TINY = np.float32(np.finfo(np.float32).tiny)


def _rotl(x, r):
    return (x << np.uint32(r)) | (x >> np.uint32(32 - r))


def _gumbel_bits(flat_u32):
    t0 = flat_u32 + _KS[1]
    x0 = t0
    x1 = _rotl(t0, 13) ^ t0
    first = True
    for g in range(5):
        for r in _ROT[g][1 if first else 0:]:
            x0 = x0 + x1
            x1 = _rotl(x1, r) ^ x0
        first = False
        x0 = x0 + _KS[(g + 1) % 3]
        x1 = x1 + _KS[(g + 2) % 3] + np.uint32(g + 1)
    return x0 ^ x1


def _body(lg_ref, samp_ref, lp_ref):
    i = pl.program_id(0)
    lg = lg_ref[...]

    col = lax.broadcasted_iota(jnp.int32, (_R, _V), 1)
    row = lax.broadcasted_iota(jnp.int32, (_R, _V), 0) + i * _R
    flat = (row * _V + col).astype(jnp.uint32)

    bits = _gumbel_bits(flat)
    fl = lax.bitcast_convert_type(
        (bits >> np.uint32(9)) | np.uint32(0x3F800000), jnp.float32) - 1.0
    u = jnp.maximum(fl + _TINY, _TINY)
    gum = -jnp.log(-jnp.log(u))
    t = gum + lg

    m_t = jnp.max(t, axis=-1, keepdims=True)
    samp = jnp.min(jnp.where(t == m_t, col, _V), axis=-1, keepdims=True)

    m_l = jnp.max(lg, axis=-1, keepdims=True)
    s = jnp.sum(jnp.exp(lg - m_l), axis=-1, keepdims=True)
    chosen = jnp.sum(jnp.where(col == samp, lg, 0.0), axis=-1, keepdims=True)
    lp_row = (chosen - m_l) - jnp.log(s)

    samp_ref[0] = samp
    lp_ref[0] = jnp.full((_R, 1), jnp.sum(lp_row), jnp.float32)


def kernel(logits):
    lg = logits.reshape(_NROWS, _V)
    nblk = _NROWS // _R
    samp, lp = pl.pallas_call(
        _body,
        grid=(nblk,),
        in_specs=[pl.BlockSpec((_R, _V), lambda i: (i, 0))],
        out_specs=[
            pl.BlockSpec((1, _R, 1), lambda i: (i, 0, 0)),
            pl.BlockSpec((1, _R, 1), lambda i: (i, 0, 0)),
        ],
        out_shape=[
            jax.ShapeDtypeStruct((nblk, _R, 1), jnp.int32),
            jax.ShapeDtypeStruct((nblk, _R, 1), jnp.float32),
        ],
    )(lg)
    sample = samp[..., 0]
    log_prob = lp[:, 0, 0]
    return sample, log_prob
